# baseline (speedup 1.0000x reference)
import jax
import jax.numpy as jnp
from jax.experimental import pallas as pl
from jax.experimental.pallas import tpu as pltpu

_NUM_SAMPLES = 512
_N_BISECT = 32
_BB = 8
_BD = 256


def _fused_kernel(x_ref, o_ref, y_scr):
    j = pl.program_id(1)

    @pl.when(j == 0)
    def _compute_stats():
        x = x_ref[...]
        d = x.shape[1]
        target = jnp.float32(d - d * 3 // 4)

        lo = jnp.min(x, axis=1, keepdims=True)
        hi = jnp.max(x, axis=1, keepdims=True)

        def body(_, carry):
            lo, hi = carry
            mid = lo * 0.5 + hi * 0.5
            c = jnp.sum((x <= mid).astype(jnp.float32), axis=1, keepdims=True)
            pred = c >= target
            return jnp.where(pred, lo, mid), jnp.where(pred, mid, hi)

        lo, hi = jax.lax.fori_loop(0, _N_BISECT, body, (lo, hi))
        mean = hi

        mu = jnp.sum(x, axis=1, keepdims=True) / d
        var = jnp.sum((x - mu) ** 2, axis=1, keepdims=True) / (d - 1)
        std = jnp.sqrt(var)

        x_norm = (x - mean) / std ** 0.3
        expo = jnp.clip(-x_norm / 0.001, -50.0, 50.0)
        y_scr[...] = 1.0 / (1.0 + jnp.exp(expo))

    yb = y_scr[:, pl.ds(j * _BD, _BD)]
    o_ref[...] = jnp.broadcast_to(yb[..., None], o_ref.shape)


def kernel(x, k):
    del k
    b, d = x.shape

    out = pl.pallas_call(
        _fused_kernel,
        grid=(b // _BB, d // _BD),
        in_specs=[pl.BlockSpec((_BB, d), lambda i, j: (i, 0))],
        out_specs=pl.BlockSpec((_BB, _BD, _NUM_SAMPLES), lambda i, j: (i, j, 0)),
        out_shape=jax.ShapeDtypeStruct((b, d, _NUM_SAMPLES), x.dtype),
        scratch_shapes=[pltpu.VMEM((_BB, d), jnp.float32)],
    )(x)
    return out


# speedup vs baseline: 1.1196x; 1.1196x over previous
import jax
import jax.numpy as jnp
from jax.experimental import pallas as pl

_NUM_SAMPLES = 512
_N_BISECT = 32
_BB = 8
_BD = 512


def _stats_kernel(x_ref, y_ref):
    x = x_ref[...]
    b, d = x.shape
    target = jnp.float32(d - d * 3 // 4)

    lo = jnp.min(x, axis=1, keepdims=True)
    hi = jnp.max(x, axis=1, keepdims=True)

    def body(_, carry):
        lo, hi = carry
        mid = lo * 0.5 + hi * 0.5
        c = jnp.sum((x <= mid).astype(jnp.float32), axis=1, keepdims=True)
        pred = c >= target
        return jnp.where(pred, lo, mid), jnp.where(pred, mid, hi)

    lo, hi = jax.lax.fori_loop(0, _N_BISECT, body, (lo, hi))
    mean = hi

    mu = jnp.sum(x, axis=1, keepdims=True) / d
    var = jnp.sum((x - mu) ** 2, axis=1, keepdims=True) / (d - 1)
    std = jnp.sqrt(var)

    x_norm = (x - mean) / std ** 0.3
    expo = jnp.clip(-x_norm / 0.001, -50.0, 50.0)
    y_ref[...] = 1.0 / (1.0 + jnp.exp(expo))


def _bcast_kernel(y_ref, o_ref):
    o_ref[...] = jnp.broadcast_to(y_ref[...][..., None], o_ref.shape)


def kernel(x, k):
    del k
    b, d = x.shape

    y = pl.pallas_call(
        _stats_kernel,
        out_shape=jax.ShapeDtypeStruct((b, d), x.dtype),
    )(x)

    out = pl.pallas_call(
        _bcast_kernel,
        grid=(b // _BB, d // _BD),
        in_specs=[pl.BlockSpec((_BB, _BD), lambda i, j: (i, j))],
        out_specs=pl.BlockSpec((_BB, _BD, _NUM_SAMPLES), lambda i, j: (i, j, 0)),
        out_shape=jax.ShapeDtypeStruct((b, d, _NUM_SAMPLES), x.dtype),
    )(y)
    return out


# speedup vs baseline: 1.1689x; 1.0440x over previous
import jax
import jax.numpy as jnp
from jax.experimental import pallas as pl

_NUM_SAMPLES = 512
_N_BISECT = 32
_BB = 8
_BD = 256


def _stats_kernel(x_ref, y_ref):
    x = x_ref[...]
    b, d = x.shape
    target = jnp.float32(d - d * 3 // 4)

    lo = jnp.min(x, axis=1, keepdims=True)
    hi = jnp.max(x, axis=1, keepdims=True)

    def body(_, carry):
        lo, hi = carry
        mid = lo * 0.5 + hi * 0.5
        c = jnp.sum((x <= mid).astype(jnp.float32), axis=1, keepdims=True)
        pred = c >= target
        return jnp.where(pred, lo, mid), jnp.where(pred, mid, hi)

    lo, hi = jax.lax.fori_loop(0, _N_BISECT, body, (lo, hi))
    mean = hi

    mu = jnp.sum(x, axis=1, keepdims=True) / d
    var = jnp.sum((x - mu) ** 2, axis=1, keepdims=True) / (d - 1)
    std = jnp.sqrt(var)

    x_norm = (x - mean) / std ** 0.3
    expo = jnp.clip(-x_norm / 0.001, -50.0, 50.0)
    y_ref[...] = 1.0 / (1.0 + jnp.exp(expo))


def _bcast_kernel(y_ref, o_ref):
    o_ref[...] = jnp.broadcast_to(y_ref[...][..., None], o_ref.shape)


def kernel(x, k):
    del k
    b, d = x.shape

    y = pl.pallas_call(
        _stats_kernel,
        out_shape=jax.ShapeDtypeStruct((b, d), x.dtype),
    )(x)

    out = pl.pallas_call(
        _bcast_kernel,
        grid=(b // _BB, d // _BD),
        in_specs=[pl.BlockSpec((_BB, _BD), lambda i, j: (i, j))],
        out_specs=pl.BlockSpec((_BB, _BD, _NUM_SAMPLES), lambda i, j: (i, j, 0)),
        out_shape=jax.ShapeDtypeStruct((b, d, _NUM_SAMPLES), x.dtype),
    )(y)
    return out
